# baseline (speedup 1.0000x reference)
import jax
import jax.numpy as jnp
from jax import lax
from jax.experimental import pallas as pl
from jax.experimental.pallas import tpu as pltpu
from jax.experimental.pallas import tpu_sc as plsc

N_NODES = 100000
NW = 32
G = 8
C_NODES = 16 * G
CHUNKS = 26
PAIRS = CHUNKS // 2
TILE_NODES = C_NODES * CHUNKS
N_PAD = TILE_NODES * NW
SEGS = 4 * G
GROWS = 512 * G


def _sc_body(table_hbm, idx_hbm, ox_hbm, oy_hbm, oz_hbm,
             idx0, idx1, gb0, gb1,
             obx0, oby0, obz0, obx1, oby1, obz1,
             sg0, sg1, so0, so1):
    wid = lax.axis_index("s") * 2 + lax.axis_index("c")
    lane = lax.iota(jnp.int32, 16)
    lane_off = lane * 32
    cols = [jnp.full((16,), c, jnp.int32) for c in range(4)]

    def fire(c, idx_v, gbuf, sem):
        seg_base = (wid * CHUNKS + c) * SEGS
        pltpu.sync_copy(idx_hbm.at[pl.ds(seg_base, SEGS)], idx_v)
        for s in range(SEGS):
            pltpu.async_copy(table_hbm.at[idx_v.at[s]],
                             gbuf.at[pl.ds(s * 128, 128)], sem)

    def drain_gather(idx_v, gbuf, sem):
        for s in range(SEGS):
            pltpu.make_async_copy(table_hbm.at[idx_v.at[s]],
                                  gbuf.at[pl.ds(s * 128, 128)], sem).wait()

    def drain_out(obx, oby, obz, sem):
        pltpu.make_async_copy(obx, ox_hbm.at[pl.ds(0, C_NODES)], sem).wait()
        pltpu.make_async_copy(oby, oy_hbm.at[pl.ds(0, C_NODES)], sem).wait()
        pltpu.make_async_copy(obz, oz_hbm.at[pl.ds(0, C_NODES)], sem).wait()

    def compute(c, gbuf, obx, oby, obz, sem):
        def group_body(g, carry2):
            rowbase = g * 512
            zero = jnp.zeros((16,), jnp.float32)
            sxx = sxy = sxz = syy = syz = szz = dx_ = dy_ = dz_ = zero
            for k in range(16):
                r1 = lane_off + (rowbase + 2 * k)
                r2 = r1 + 1
                x1 = plsc.load_gather(gbuf, [r1, cols[0]])
                y1 = plsc.load_gather(gbuf, [r1, cols[1]])
                z1 = plsc.load_gather(gbuf, [r1, cols[2]])
                u1 = plsc.load_gather(gbuf, [r1, cols[3]])
                x2 = plsc.load_gather(gbuf, [r2, cols[0]])
                y2 = plsc.load_gather(gbuf, [r2, cols[1]])
                z2 = plsc.load_gather(gbuf, [r2, cols[2]])
                u2 = plsc.load_gather(gbuf, [r2, cols[3]])
                dx = x1 - x2
                dy = y1 - y2
                dz = z1 - z2
                du = u1 - u2
                r2sq = dx * dx + dy * dy + dz * dz
                w2 = jnp.where(r2sq == 0.0, jnp.float32(1.0), 1.0 / r2sq)
                t = w2 * dx
                s = w2 * dy
                r = w2 * dz
                sxx = sxx + t * dx
                sxy = sxy + t * dy
                sxz = sxz + t * dz
                dx_ = dx_ + t * du
                syy = syy + s * dy
                syz = syz + s * dz
                dy_ = dy_ + s * du
                szz = szz + r * dz
                dz_ = dz_ + r * du
            m0 = syy * szz - syz * syz
            m1 = sxy * szz - syz * sxz
            m2 = sxy * syz - syy * sxz
            det_a = sxx * m0 - sxy * m1 + sxz * m2
            nx = dx_ * m0 - dy_ * m1 + dz_ * m2
            ny = (sxx * (dy_ * szz - dz_ * syz)
                  - sxy * (dx_ * szz - dz_ * sxz)
                  + sxz * (dx_ * syz - dy_ * sxz))
            nz = (sxx * (syy * dz_ - syz * dy_)
                  - sxy * (sxy * dz_ - syz * dx_)
                  + sxz * (sxy * dy_ - syy * dx_))
            inv_det = 1.0 / det_a
            obx[pl.ds(g * 16, 16)] = nx * inv_det
            oby[pl.ds(g * 16, 16)] = ny * inv_det
            obz[pl.ds(g * 16, 16)] = nz * inv_det
            return carry2

        lax.fori_loop(0, G, group_body, 0)
        out_base = wid * TILE_NODES + c * C_NODES
        pltpu.async_copy(obx, ox_hbm.at[pl.ds(out_base, C_NODES)], sem)
        pltpu.async_copy(oby, oy_hbm.at[pl.ds(out_base, C_NODES)], sem)
        pltpu.async_copy(obz, oz_hbm.at[pl.ds(out_base, C_NODES)], sem)

    fire(0, idx0, gb0, sg0)

    def pair_body(p, carry):
        c0 = 2 * p
        fire(c0 + 1, idx1, gb1, sg1)
        drain_gather(idx0, gb0, sg0)

        @pl.when(p > 0)
        def _():
            drain_out(obx0, oby0, obz0, so0)

        compute(c0, gb0, obx0, oby0, obz0, so0)

        @pl.when(p + 1 < PAIRS)
        def _():
            fire(c0 + 2, idx0, gb0, sg0)

        drain_gather(idx1, gb1, sg1)

        @pl.when(p > 0)
        def _():
            drain_out(obx1, oby1, obz1, so1)

        compute(c0 + 1, gb1, obx1, oby1, obz1, so1)
        return carry

    lax.fori_loop(0, PAIRS, pair_body, 0)
    drain_out(obx0, oby0, obz0, so0)
    drain_out(obx1, oby1, obz1, so1)


@jax.jit
def _run(table, idx2d):
    f32 = jnp.float32
    out = jax.ShapeDtypeStruct((N_PAD,), f32)
    call = pl.kernel(
        _sc_body,
        out_type=[out, out, out],
        mesh=plsc.VectorSubcoreMesh(core_axis_name="c", subcore_axis_name="s"),
        scratch_types=[
            pltpu.VMEM((SEGS, 128), jnp.int32),
            pltpu.VMEM((SEGS, 128), jnp.int32),
            pltpu.VMEM((GROWS, 4), f32),
            pltpu.VMEM((GROWS, 4), f32),
            pltpu.VMEM((C_NODES,), f32),
            pltpu.VMEM((C_NODES,), f32),
            pltpu.VMEM((C_NODES,), f32),
            pltpu.VMEM((C_NODES,), f32),
            pltpu.VMEM((C_NODES,), f32),
            pltpu.VMEM((C_NODES,), f32),
            pltpu.SemaphoreType.DMA,
            pltpu.SemaphoreType.DMA,
            pltpu.SemaphoreType.DMA,
            pltpu.SemaphoreType.DMA,
        ],
        compiler_params=pltpu.CompilerParams(
            needs_layout_passes=False, use_tc_tiling_on_sc=False),
    )
    return call(table, idx2d)


def kernel(coordinates, u, connectivity_tensor):
    n = coordinates.shape[0]
    table = jnp.concatenate([coordinates, u], axis=1)
    idx = connectivity_tensor.reshape(n, 32).astype(jnp.int32)
    idx = jnp.pad(idx, ((0, N_PAD - n), (0, 0)))
    idx2d = idx.reshape(N_PAD * 32 // 128, 128)
    ox, oy, oz = _run(table, idx2d)
    return (ox[:n, None], oy[:n, None], oz[:n, None])


# speedup vs baseline: 2.6608x; 2.6608x over previous
import jax
import jax.numpy as jnp
from jax import lax
from jax.experimental import pallas as pl
from jax.experimental.pallas import tpu as pltpu
from jax.experimental.pallas import tpu_sc as plsc

N_NODES = 100000
NW = 32
G = 8
C_NODES = 16 * G
CHUNKS = 25
TILE_NODES = C_NODES * CHUNKS
N_PAD = TILE_NODES * NW
SEGS = 4 * G
T_PAD = 100352
T_PER_TILE = T_PAD // 16


def _sc_body(table_hbm, idx_hbm, ox_hbm, oy_hbm, oz_hbm,
             table_sh, idx_v, gbuf, obx, oby, obz, sem):
    cid = lax.axis_index("c")
    sid = lax.axis_index("s")
    wid = sid * 2 + cid
    lane = lax.iota(jnp.int32, 16)
    lane_off = lane * 32
    cols = [jnp.full((16,), c, jnp.int32) for c in range(4)]

    trow = sid * T_PER_TILE
    pltpu.sync_copy(table_hbm.at[pl.ds(trow, T_PER_TILE)],
                    table_sh.at[pl.ds(trow, T_PER_TILE)])
    plsc.subcore_barrier()

    def chunk_body(c, carry):
        seg_base = (wid * CHUNKS + c) * SEGS
        pltpu.sync_copy(idx_hbm.at[pl.ds(seg_base, SEGS)], idx_v)
        cps = [
            pltpu.async_copy(table_sh.at[idx_v.at[s]],
                             gbuf.at[pl.ds(s * 128, 128)], sem)
            for s in range(SEGS)
        ]
        for cp in cps:
            cp.wait()

        def group_body(g, carry2):
            rowbase = g * 512
            zero = jnp.zeros((16,), jnp.float32)
            sxx = sxy = sxz = syy = syz = szz = dx_ = dy_ = dz_ = zero
            for k in range(16):
                r1 = lane_off + (rowbase + 2 * k)
                r2 = r1 + 1
                x1 = plsc.load_gather(gbuf, [r1, cols[0]])
                y1 = plsc.load_gather(gbuf, [r1, cols[1]])
                z1 = plsc.load_gather(gbuf, [r1, cols[2]])
                u1 = plsc.load_gather(gbuf, [r1, cols[3]])
                x2 = plsc.load_gather(gbuf, [r2, cols[0]])
                y2 = plsc.load_gather(gbuf, [r2, cols[1]])
                z2 = plsc.load_gather(gbuf, [r2, cols[2]])
                u2 = plsc.load_gather(gbuf, [r2, cols[3]])
                dx = x1 - x2
                dy = y1 - y2
                dz = z1 - z2
                du = u1 - u2
                r2sq = dx * dx + dy * dy + dz * dz
                w2 = jnp.where(r2sq == 0.0, jnp.float32(1.0), 1.0 / r2sq)
                t = w2 * dx
                s = w2 * dy
                r = w2 * dz
                sxx = sxx + t * dx
                sxy = sxy + t * dy
                sxz = sxz + t * dz
                dx_ = dx_ + t * du
                syy = syy + s * dy
                syz = syz + s * dz
                dy_ = dy_ + s * du
                szz = szz + r * dz
                dz_ = dz_ + r * du
            m0 = syy * szz - syz * syz
            m1 = sxy * szz - syz * sxz
            m2 = sxy * syz - syy * sxz
            det_a = sxx * m0 - sxy * m1 + sxz * m2
            nx = dx_ * m0 - dy_ * m1 + dz_ * m2
            ny = (sxx * (dy_ * szz - dz_ * syz)
                  - sxy * (dx_ * szz - dz_ * sxz)
                  + sxz * (dx_ * syz - dy_ * sxz))
            nz = (sxx * (syy * dz_ - syz * dy_)
                  - sxy * (sxy * dz_ - syz * dx_)
                  + sxz * (sxy * dy_ - syy * dx_))
            inv_det = 1.0 / det_a
            obx[pl.ds(g * 16, 16)] = nx * inv_det
            oby[pl.ds(g * 16, 16)] = ny * inv_det
            obz[pl.ds(g * 16, 16)] = nz * inv_det
            return carry2

        lax.fori_loop(0, G, group_body, 0)
        out_base = wid * TILE_NODES + c * C_NODES
        pltpu.sync_copy(obx, ox_hbm.at[pl.ds(out_base, C_NODES)])
        pltpu.sync_copy(oby, oy_hbm.at[pl.ds(out_base, C_NODES)])
        pltpu.sync_copy(obz, oz_hbm.at[pl.ds(out_base, C_NODES)])
        return carry

    lax.fori_loop(0, CHUNKS, chunk_body, 0)


@jax.jit
def _run(table, idx2d):
    f32 = jnp.float32
    out = jax.ShapeDtypeStruct((N_PAD,), f32)
    call = pl.kernel(
        _sc_body,
        out_type=[out, out, out],
        mesh=plsc.VectorSubcoreMesh(core_axis_name="c", subcore_axis_name="s"),
        scratch_types=[
            pltpu.VMEM_SHARED((T_PAD, 4), f32),
            pltpu.VMEM((SEGS, 128), jnp.int32),
            pltpu.VMEM((512 * G, 4), f32),
            pltpu.VMEM((C_NODES,), f32),
            pltpu.VMEM((C_NODES,), f32),
            pltpu.VMEM((C_NODES,), f32),
            pltpu.SemaphoreType.DMA,
        ],
        compiler_params=pltpu.CompilerParams(
            needs_layout_passes=False, use_tc_tiling_on_sc=False),
    )
    return call(table, idx2d)


def kernel(coordinates, u, connectivity_tensor):
    n = coordinates.shape[0]
    table = jnp.concatenate([coordinates, u], axis=1)
    table = jnp.pad(table, ((0, T_PAD - n), (0, 0)))
    idx = connectivity_tensor.reshape(n, 32).astype(jnp.int32)
    idx = jnp.pad(idx, ((0, N_PAD - n), (0, 0)))
    idx2d = idx.reshape(N_PAD * 32 // 128, 128)
    ox, oy, oz = _run(table, idx2d)
    return (ox[:n, None], oy[:n, None], oz[:n, None])
